# Initial kernel scaffold; baseline (speedup 1.0000x reference)
import jax
import jax.numpy as jnp
from jax.experimental import pallas as pl


def kernel(x, edge_index, Wl0, bl0, Wr0, Wl1, bl1, Wr1, W, b):
    raise NotImplementedError("write your pallas kernel here")


# speedup vs baseline: 1.0200x; 1.0200x over previous
import functools

import jax
import jax.numpy as jnp
from jax.experimental import pallas as pl

N = 10000
D = 128
_ROWS = 1000


def _fused_lin_body(a_ref, x_ref, wl_ref, bl_ref, wr_ref, o_ref, *, relu):
    acc = jax.lax.dot_general(a_ref[...], wl_ref[...], (((1,), (1,)), ((), ())),
                              preferred_element_type=jnp.float32)
    acc += jax.lax.dot_general(x_ref[...], wr_ref[...], (((1,), (1,)), ((), ())),
                               preferred_element_type=jnp.float32)
    acc += bl_ref[...]
    o_ref[...] = jnp.maximum(acc, 0.0) if relu else acc


def _fused_lin(a, x, wl, bl, wr, relu):
    grid = (N // _ROWS,)
    return pl.pallas_call(
        functools.partial(_fused_lin_body, relu=relu),
        grid=grid,
        in_specs=[
            pl.BlockSpec((_ROWS, D), lambda i: (i, 0)),
            pl.BlockSpec((_ROWS, D), lambda i: (i, 0)),
            pl.BlockSpec((D, D), lambda i: (0, 0)),
            pl.BlockSpec((1, D), lambda i: (0, 0)),
            pl.BlockSpec((D, D), lambda i: (0, 0)),
        ],
        out_specs=pl.BlockSpec((_ROWS, D), lambda i: (i, 0)),
        out_shape=jax.ShapeDtypeStruct((N, D), jnp.float32),
    )(a, x, wl, bl.reshape(1, D), wr)


def _segmax(x, src, dst):
    msg = jnp.take(x, src, axis=0)
    aggr = jax.ops.segment_max(msg, dst, num_segments=N)
    return jnp.where(jnp.isfinite(aggr), aggr, 0.0)


def kernel(x, edge_index, Wl0, bl0, Wr0, Wl1, bl1, Wr1, W, b):
    src = edge_index[0].astype(jnp.int32)
    dst = edge_index[1].astype(jnp.int32)
    h = _fused_lin(_segmax(x, src, dst), x, Wl0, bl0, Wr0, relu=True)
    h = _fused_lin(_segmax(h, src, dst), h, Wl1, bl1, Wr1, relu=True)
    out = _fused_lin(h, h, W, b, jnp.zeros_like(W), relu=False)
    return out


# speedup vs baseline: 1.8253x; 1.7896x over previous
import functools

import jax
import jax.numpy as jnp
from jax import lax
from jax.experimental import pallas as pl
from jax.experimental.pallas import tpu as pltpu
from jax.experimental.pallas import tpu_sc as plsc

N = 10000
E = 320000
D = 128
L = 16
NC = 2
NS = 16
NW = NC * NS
NB = 313
NPAD = NW * NB
K = 512
KSUB = 128
SB = 24
CHUNK = 4000
NCHUNK = E // CHUNK
CAPR = E + K
SHIFT = 14
NEG = float("-inf")

_mesh = lambda: plsc.VectorSubcoreMesh(core_axis_name="c", subcore_axis_name="s")


def _wid():
    return lax.axis_index("s") * NC + lax.axis_index("c")


def _part_body(src_hbm, dst_hbm, pkl, nblk, srcb0, dstb0, srcb1, dstb1, pkb,
               cnt_v, sem0, sem1):
    w = _wid()
    lo = w * NB
    hi = lo + NB
    lanes = lax.iota(jnp.int32, L)

    def _fire(ci, sb, db, sem):
        off = pl.multiple_of(ci * CHUNK, 8)
        pltpu.async_copy(src_hbm.at[pl.ds(off, CHUNK)], sb, sem)
        pltpu.async_copy(dst_hbm.at[pl.ds(off, CHUNK)], db, sem)

    def _wait(ci, sb, db, sem):
        off = pl.multiple_of(ci * CHUNK, 8)
        pltpu.make_async_copy(src_hbm.at[pl.ds(off, CHUNK)], sb, sem).wait()
        pltpu.make_async_copy(dst_hbm.at[pl.ds(off, CHUNK)], db, sem).wait()

    def _scan(sb, db, carry):
        off_v, g = carry

        def scan_vreg(i, off_v):
            dd = db[pl.ds(i * L, L)]
            ss = sb[pl.ds(i * L, L)]
            m = (dd >= lo) & (dd < hi)
            ones = jnp.where(m, 1, 0).astype(jnp.int32)
            pos = off_v + plsc.cumsum(ones) - 1
            pk = ss | ((dd - lo) << SHIFT)
            plsc.store_scatter(pkb, [pos], pk, mask=m)
            return off_v + plsc.all_reduce_population_count(m)

        off_v = lax.fori_loop(0, CHUNK // L, scan_vreg, off_v)
        off_s = jnp.max(off_v)
        nb = off_s // K

        def flush(b, _):
            pltpu.sync_copy(pkb.at[pl.ds(b * K, K)],
                            pkl.at[pl.ds(pl.multiple_of(w * CAPR + g + b * K, 8), K)])
            return 0

        lax.fori_loop(0, nb, flush, 0)
        r = off_s - nb * K
        nmv = jnp.where(nb > 0, (r + L - 1) // L, 0)

        def mv(j, _):
            pkb[pl.ds(j * L, L)] = plsc.load_gather(pkb, [nb * K + j * L + lanes])
            return 0

        lax.fori_loop(0, nmv, mv, 0)
        return off_v - nb * K, g + nb * K

    _fire(0, srcb0, dstb0, sem0)

    def chunk_pair(i, carry):
        ca = 2 * i
        _wait(ca, srcb0, dstb0, sem0)
        _fire(ca + 1, srcb1, dstb1, sem1)
        carry = _scan(srcb0, dstb0, carry)
        _wait(ca + 1, srcb1, dstb1, sem1)

        @pl.when(ca + 2 < NCHUNK)
        def _():
            _fire(ca + 2, srcb0, dstb0, sem0)

        carry = _scan(srcb1, dstb1, carry)
        return carry

    off0 = jnp.zeros((L,), jnp.int32)
    off_v, g = lax.fori_loop(0, NCHUNK // 2, chunk_pair, (off0, jnp.int32(0)))
    r = jnp.max(off_v)
    dummy = jnp.full((L,), NB << SHIFT, jnp.int32)
    for j in range(K // L):
        plsc.store_scatter(pkb, [r + j * L + lanes], dummy)

    def fflush(b, _):
        pltpu.sync_copy(pkb.at[pl.ds(0, K)],
                        pkl.at[pl.ds(pl.multiple_of(w * CAPR + g, 8), K)])
        return 0

    nfin = jnp.where(r > 0, 1, 0)
    lax.fori_loop(0, nfin, fflush, 0)
    cnt_v[...] = jnp.zeros((L,), jnp.int32) + (g // K + nfin)
    pltpu.sync_copy(cnt_v, nblk.at[pl.ds(pl.multiple_of(w * L, 8), L)])


def _partition(src, dst):
    return pl.kernel(
        _part_body,
        out_type=[
            jax.ShapeDtypeStruct((NW * CAPR,), jnp.int32),
            jax.ShapeDtypeStruct((NW * L,), jnp.int32),
        ],
        mesh=_mesh(),
        compiler_params=pltpu.CompilerParams(needs_layout_passes=False),
        scratch_types=[
            pltpu.VMEM((CHUNK,), jnp.int32),
            pltpu.VMEM((CHUNK,), jnp.int32),
            pltpu.VMEM((CHUNK,), jnp.int32),
            pltpu.VMEM((CHUNK,), jnp.int32),
            pltpu.VMEM((CHUNK + K + L,), jnp.int32),
            pltpu.VMEM((L,), jnp.int32),
            pltpu.SemaphoreType.DMA,
            pltpu.SemaphoreType.DMA,
        ],
    )(src, dst)


def _seg_body(x_hbm, pkl, nblk, outf, acc1, pk_stage, pk_v, idx_v, rows_v,
              cnt_v, sem, *, fixup):
    w = _wid()
    pltpu.sync_copy(nblk.at[pl.ds(pl.multiple_of(w * L, 8), L)], cnt_v)
    nb = jnp.max(cnt_v[...])

    init = NEG if fixup else 0.0

    def ini(i, _):
        acc1[pl.ds(i * L, L)] = jnp.full((L,), init, jnp.float32)
        return 0

    lax.fori_loop(0, (NB + 1) * D // L, ini, 0)

    pltpu.sync_copy(pkl.at[pl.ds(pl.multiple_of(w * CAPR, 8), SB * K)], pk_stage)

    lanes = lax.iota(jnp.int32, L)
    zero_v = jnp.zeros((L,), jnp.int32)

    def _process(pkref, boff):
        def unp(j, _):
            v = pkref[pl.ds(boff + j * L, L)]
            idx_v[pl.ds(j * L, L)] = v & ((1 << SHIFT) - 1)
            return 0

        lax.fori_loop(0, K // L, unp, 0)
        cps = [
            pltpu.async_copy(x_hbm.at[idx_v.at[pl.ds(j * KSUB, KSUB)]],
                             rows_v.at[pl.ds(j * KSUB, KSUB)], sem)
            for j in range(K // KSUB)
        ]
        for cp in cps:
            cp.wait()

        def edge(k, _):
            lsp = plsc.load_gather(pkref, [boff + k + zero_v]) >> SHIFT
            base = lsp * D
            for f in range(D // L):
                ii = base + f * L + lanes
                a = plsc.load_gather(acc1, [ii])
                rr = rows_v[k, pl.ds(f * L, L)]
                plsc.store_scatter(acc1, [ii], jnp.maximum(a, rr))
            return 0

        lax.fori_loop(0, K, edge, 0)
        return 0

    def batch_staged(b, _):
        return _process(pk_stage, b * K)

    lax.fori_loop(0, jnp.minimum(nb, SB), batch_staged, 0)

    def batch_hbm(b, _):
        pltpu.sync_copy(pkl.at[pl.ds(pl.multiple_of(w * CAPR + b * K, 8), K)],
                        pk_v)
        return _process(pk_v, 0)

    lax.fori_loop(SB, nb, batch_hbm, 0)

    if fixup:
        def fix(i, _):
            v = acc1[pl.ds(i * L, L)]
            acc1[pl.ds(i * L, L)] = jnp.where(v == NEG, 0.0, v)
            return 0

        lax.fori_loop(0, NB * D // L, fix, 0)

    pltpu.sync_copy(acc1.at[pl.ds(0, NB * D)],
                    outf.at[pl.ds(pl.multiple_of(w * NB * D, 8), NB * D)])


def _segmax(x2d, pkl, nblk, fixup):
    out = pl.kernel(
        functools.partial(_seg_body, fixup=fixup),
        out_type=jax.ShapeDtypeStruct((NPAD * D,), jnp.float32),
        mesh=_mesh(),
        compiler_params=pltpu.CompilerParams(needs_layout_passes=False),
        scratch_types=[
            pltpu.VMEM(((NB + 1) * D,), jnp.float32),
            pltpu.VMEM((SB * K,), jnp.int32),
            pltpu.VMEM((K,), jnp.int32),
            pltpu.VMEM((K,), jnp.int32),
            pltpu.VMEM((K, D), jnp.float32),
            pltpu.VMEM((L,), jnp.int32),
            pltpu.SemaphoreType.DMA,
        ],
    )(x2d, pkl, nblk)
    return out.reshape(NPAD, D)


_ROWS = 1000


def _lin_body(a_ref, x_ref, wl_ref, bl_ref, wr_ref, o_ref):
    acc = lax.dot_general(a_ref[...], wl_ref[...], (((1,), (1,)), ((), ())),
                          preferred_element_type=jnp.float32)
    acc += lax.dot_general(x_ref[...], wr_ref[...], (((1,), (1,)), ((), ())),
                           preferred_element_type=jnp.float32)
    acc += bl_ref[...]
    o_ref[...] = jnp.maximum(acc, 0.0)


def _fused_lin(a, x, wl, bl, wr):
    return pl.pallas_call(
        _lin_body,
        grid=(N // _ROWS,),
        in_specs=[
            pl.BlockSpec((_ROWS, D), lambda i: (i, 0)),
            pl.BlockSpec((_ROWS, D), lambda i: (i, 0)),
            pl.BlockSpec((D, D), lambda i: (0, 0)),
            pl.BlockSpec((1, D), lambda i: (0, 0)),
            pl.BlockSpec((D, D), lambda i: (0, 0)),
        ],
        out_specs=pl.BlockSpec((_ROWS, D), lambda i: (i, 0)),
        out_shape=jax.ShapeDtypeStruct((N, D), jnp.float32),
    )(a, x, wl, bl.reshape(1, D), wr)


def _final_body(a_ref, x_ref, wl_ref, bl_ref, wr_ref, wf_ref, bf_ref, o_ref):
    acc = lax.dot_general(a_ref[...], wl_ref[...], (((1,), (1,)), ((), ())),
                          preferred_element_type=jnp.float32)
    acc += lax.dot_general(x_ref[...], wr_ref[...], (((1,), (1,)), ((), ())),
                           preferred_element_type=jnp.float32)
    h = jnp.maximum(acc + bl_ref[...], 0.0)
    o_ref[...] = lax.dot_general(h, wf_ref[...], (((1,), (1,)), ((), ())),
                                 preferred_element_type=jnp.float32) + bf_ref[...]


def _final_lin(a, x, wl, bl, wr, wf, bf):
    return pl.pallas_call(
        _final_body,
        grid=(N // _ROWS,),
        in_specs=[
            pl.BlockSpec((_ROWS, D), lambda i: (i, 0)),
            pl.BlockSpec((_ROWS, D), lambda i: (i, 0)),
            pl.BlockSpec((D, D), lambda i: (0, 0)),
            pl.BlockSpec((1, D), lambda i: (0, 0)),
            pl.BlockSpec((D, D), lambda i: (0, 0)),
            pl.BlockSpec((D, D), lambda i: (0, 0)),
            pl.BlockSpec((1, D), lambda i: (0, 0)),
        ],
        out_specs=pl.BlockSpec((_ROWS, D), lambda i: (i, 0)),
        out_shape=jax.ShapeDtypeStruct((N, D), jnp.float32),
    )(a, x, wl, bl.reshape(1, D), wr, wf, bf.reshape(1, D))


def kernel(x, edge_index, Wl0, bl0, Wr0, Wl1, bl1, Wr1, W, b):
    src = edge_index[0].astype(jnp.int32)
    dst = edge_index[1].astype(jnp.int32)
    pkl, nblk = _partition(src, dst)
    a0 = _segmax(x, pkl, nblk, fixup=True)
    h1 = _fused_lin(a0, x, Wl0, bl0, Wr0)
    a1 = _segmax(h1, pkl, nblk, fixup=False)
    out = _final_lin(a1, h1, Wl1, bl1, Wr1, W, b)
    return out


# speedup vs baseline: 2.5746x; 1.4105x over previous
import functools

import jax
import jax.numpy as jnp
from jax import lax
from jax.experimental import pallas as pl
from jax.experimental.pallas import tpu as pltpu
from jax.experimental.pallas import tpu_sc as plsc

N = 10000
E = 320000
D = 128
L = 16
NC = 2
NS = 16
NW = NC * NS
NB = 313
NPAD = NW * NB
K = 512
KSUB = 128
SB = 24
CHUNK = 4000
NCHUNK = E // CHUNK
CAPR = E + K
SHIFT = 14
NEG = float("-inf")

_mesh = lambda: plsc.VectorSubcoreMesh(core_axis_name="c", subcore_axis_name="s")


def _wid():
    return lax.axis_index("s") * NC + lax.axis_index("c")


def _part_body(src_hbm, dst_hbm, pkl, nblk, srcb0, dstb0, srcb1, dstb1, pkb,
               cnt_v, sem0, sem1):
    w = _wid()
    lo = w * NB
    hi = lo + NB
    lanes = lax.iota(jnp.int32, L)

    def _fire(ci, sb, db, sem):
        off = pl.multiple_of(ci * CHUNK, 8)
        pltpu.async_copy(src_hbm.at[pl.ds(off, CHUNK)], sb, sem)
        pltpu.async_copy(dst_hbm.at[pl.ds(off, CHUNK)], db, sem)

    def _wait(ci, sb, db, sem):
        off = pl.multiple_of(ci * CHUNK, 8)
        pltpu.make_async_copy(src_hbm.at[pl.ds(off, CHUNK)], sb, sem).wait()
        pltpu.make_async_copy(dst_hbm.at[pl.ds(off, CHUNK)], db, sem).wait()

    def _scan(sb, db, carry):
        off_v, g = carry

        def scan_vreg(i, off_v):
            dd = db[pl.ds(i * L, L)]
            ss = sb[pl.ds(i * L, L)]
            m = (dd >= lo) & (dd < hi)
            ones = jnp.where(m, 1, 0).astype(jnp.int32)
            pos = off_v + plsc.cumsum(ones) - 1
            pk = ss | ((dd - lo) << SHIFT)
            plsc.store_scatter(pkb, [pos], pk, mask=m)
            return off_v + plsc.all_reduce_population_count(m)

        off_v = lax.fori_loop(0, CHUNK // L, scan_vreg, off_v)
        off_s = jnp.max(off_v)
        nb = off_s // K

        def flush(b, _):
            pltpu.sync_copy(pkb.at[pl.ds(b * K, K)],
                            pkl.at[pl.ds(pl.multiple_of(w * CAPR + g + b * K, 8), K)])
            return 0

        lax.fori_loop(0, nb, flush, 0)
        r = off_s - nb * K
        nmv = jnp.where(nb > 0, (r + L - 1) // L, 0)

        def mv(j, _):
            pkb[pl.ds(j * L, L)] = plsc.load_gather(pkb, [nb * K + j * L + lanes])
            return 0

        lax.fori_loop(0, nmv, mv, 0)
        return off_v - nb * K, g + nb * K

    _fire(0, srcb0, dstb0, sem0)

    def chunk_pair(i, carry):
        ca = 2 * i
        _wait(ca, srcb0, dstb0, sem0)
        _fire(ca + 1, srcb1, dstb1, sem1)
        carry = _scan(srcb0, dstb0, carry)
        _wait(ca + 1, srcb1, dstb1, sem1)

        @pl.when(ca + 2 < NCHUNK)
        def _():
            _fire(ca + 2, srcb0, dstb0, sem0)

        carry = _scan(srcb1, dstb1, carry)
        return carry

    off0 = jnp.zeros((L,), jnp.int32)
    off_v, g = lax.fori_loop(0, NCHUNK // 2, chunk_pair, (off0, jnp.int32(0)))
    r = jnp.max(off_v)
    dummy = jnp.full((L,), NB << SHIFT, jnp.int32)
    for j in range(K // L):
        plsc.store_scatter(pkb, [r + j * L + lanes], dummy)

    def fflush(b, _):
        pltpu.sync_copy(pkb.at[pl.ds(0, K)],
                        pkl.at[pl.ds(pl.multiple_of(w * CAPR + g, 8), K)])
        return 0

    nfin = jnp.where(r > 0, 1, 0)
    lax.fori_loop(0, nfin, fflush, 0)
    cnt_v[...] = jnp.zeros((L,), jnp.int32) + (g // K + nfin)
    pltpu.sync_copy(cnt_v, nblk.at[pl.ds(pl.multiple_of(w * L, 8), L)])


def _partition(src, dst):
    return pl.kernel(
        _part_body,
        out_type=[
            jax.ShapeDtypeStruct((NW * CAPR,), jnp.int32),
            jax.ShapeDtypeStruct((NW * L,), jnp.int32),
        ],
        mesh=_mesh(),
        compiler_params=pltpu.CompilerParams(needs_layout_passes=False),
        scratch_types=[
            pltpu.VMEM((CHUNK,), jnp.int32),
            pltpu.VMEM((CHUNK,), jnp.int32),
            pltpu.VMEM((CHUNK,), jnp.int32),
            pltpu.VMEM((CHUNK,), jnp.int32),
            pltpu.VMEM((CHUNK + K + L,), jnp.int32),
            pltpu.VMEM((L,), jnp.int32),
            pltpu.SemaphoreType.DMA,
            pltpu.SemaphoreType.DMA,
        ],
    )(src, dst)


def _seg_body(x_hbm, pkl, nblk, outf, acc1, pk_stage, pk_v, idx_v, rows_v,
              cnt_v, sem, *, fixup):
    w = _wid()
    pltpu.sync_copy(nblk.at[pl.ds(pl.multiple_of(w * L, 8), L)], cnt_v)
    nb = jnp.max(cnt_v[...])

    init = NEG if fixup else 0.0

    def ini(i, _):
        acc1[pl.ds(i * L, L)] = jnp.full((L,), init, jnp.float32)
        return 0

    lax.fori_loop(0, (NB + 1) * D // L, ini, 0)

    pltpu.sync_copy(pkl.at[pl.ds(pl.multiple_of(w * CAPR, 8), SB * K)], pk_stage)

    lanes = lax.iota(jnp.int32, L)
    zero_v = jnp.zeros((L,), jnp.int32)

    def _process(pkref, boff):
        def unp(j, _):
            v = pkref[pl.ds(boff + j * L, L)]
            idx_v[pl.ds(j * L, L)] = v & ((1 << SHIFT) - 1)
            return 0

        lax.fori_loop(0, K // L, unp, 0)
        cps = [
            pltpu.async_copy(x_hbm.at[idx_v.at[pl.ds(j * KSUB, KSUB)]],
                             rows_v.at[pl.ds(j * KSUB, KSUB)], sem)
            for j in range(K // KSUB)
        ]
        for cp in cps:
            cp.wait()

        def edge(k, _):
            lsp = plsc.load_gather(pkref, [boff + k + zero_v]) >> SHIFT
            base = lsp * D
            iis = [base + f * L + lanes for f in range(D // L)]
            accs = [plsc.load_gather(acc1, [ii]) for ii in iis]
            rrs = [rows_v[k, pl.ds(f * L, L)] for f in range(D // L)]
            for ii, a, rr in zip(iis, accs, rrs):
                plsc.store_scatter(acc1, [ii], jnp.maximum(a, rr))
            return 0

        lax.fori_loop(0, K, edge, 0)
        return 0

    def batch_staged(b, _):
        return _process(pk_stage, b * K)

    lax.fori_loop(0, jnp.minimum(nb, SB), batch_staged, 0)

    def batch_hbm(b, _):
        pltpu.sync_copy(pkl.at[pl.ds(pl.multiple_of(w * CAPR + b * K, 8), K)],
                        pk_v)
        return _process(pk_v, 0)

    lax.fori_loop(SB, nb, batch_hbm, 0)

    if fixup:
        def fix(i, _):
            v = acc1[pl.ds(i * L, L)]
            acc1[pl.ds(i * L, L)] = jnp.where(v == NEG, 0.0, v)
            return 0

        lax.fori_loop(0, NB * D // L, fix, 0)

    pltpu.sync_copy(acc1.at[pl.ds(0, NB * D)],
                    outf.at[pl.ds(pl.multiple_of(w * NB * D, 8), NB * D)])


def _segmax(x2d, pkl, nblk, fixup):
    out = pl.kernel(
        functools.partial(_seg_body, fixup=fixup),
        out_type=jax.ShapeDtypeStruct((NPAD * D,), jnp.float32),
        mesh=_mesh(),
        compiler_params=pltpu.CompilerParams(needs_layout_passes=False),
        scratch_types=[
            pltpu.VMEM(((NB + 1) * D,), jnp.float32),
            pltpu.VMEM((SB * K,), jnp.int32),
            pltpu.VMEM((K,), jnp.int32),
            pltpu.VMEM((K,), jnp.int32),
            pltpu.VMEM((K, D), jnp.float32),
            pltpu.VMEM((L,), jnp.int32),
            pltpu.SemaphoreType.DMA,
        ],
    )(x2d, pkl, nblk)
    return out.reshape(NPAD, D)


_ROWS = 1000


def _lin_body(a_ref, x_ref, wl_ref, bl_ref, wr_ref, o_ref):
    acc = lax.dot_general(a_ref[...], wl_ref[...], (((1,), (1,)), ((), ())),
                          preferred_element_type=jnp.float32)
    acc += lax.dot_general(x_ref[...], wr_ref[...], (((1,), (1,)), ((), ())),
                           preferred_element_type=jnp.float32)
    acc += bl_ref[...]
    o_ref[...] = jnp.maximum(acc, 0.0)


def _fused_lin(a, x, wl, bl, wr):
    return pl.pallas_call(
        _lin_body,
        grid=(N // _ROWS,),
        in_specs=[
            pl.BlockSpec((_ROWS, D), lambda i: (i, 0)),
            pl.BlockSpec((_ROWS, D), lambda i: (i, 0)),
            pl.BlockSpec((D, D), lambda i: (0, 0)),
            pl.BlockSpec((1, D), lambda i: (0, 0)),
            pl.BlockSpec((D, D), lambda i: (0, 0)),
        ],
        out_specs=pl.BlockSpec((_ROWS, D), lambda i: (i, 0)),
        out_shape=jax.ShapeDtypeStruct((N, D), jnp.float32),
    )(a, x, wl, bl.reshape(1, D), wr)


def _final_body(a_ref, x_ref, wl_ref, bl_ref, wr_ref, wf_ref, bf_ref, o_ref):
    acc = lax.dot_general(a_ref[...], wl_ref[...], (((1,), (1,)), ((), ())),
                          preferred_element_type=jnp.float32)
    acc += lax.dot_general(x_ref[...], wr_ref[...], (((1,), (1,)), ((), ())),
                           preferred_element_type=jnp.float32)
    h = jnp.maximum(acc + bl_ref[...], 0.0)
    o_ref[...] = lax.dot_general(h, wf_ref[...], (((1,), (1,)), ((), ())),
                                 preferred_element_type=jnp.float32) + bf_ref[...]


def _final_lin(a, x, wl, bl, wr, wf, bf):
    return pl.pallas_call(
        _final_body,
        grid=(N // _ROWS,),
        in_specs=[
            pl.BlockSpec((_ROWS, D), lambda i: (i, 0)),
            pl.BlockSpec((_ROWS, D), lambda i: (i, 0)),
            pl.BlockSpec((D, D), lambda i: (0, 0)),
            pl.BlockSpec((1, D), lambda i: (0, 0)),
            pl.BlockSpec((D, D), lambda i: (0, 0)),
            pl.BlockSpec((D, D), lambda i: (0, 0)),
            pl.BlockSpec((1, D), lambda i: (0, 0)),
        ],
        out_specs=pl.BlockSpec((_ROWS, D), lambda i: (i, 0)),
        out_shape=jax.ShapeDtypeStruct((N, D), jnp.float32),
    )(a, x, wl, bl.reshape(1, D), wr, wf, bf.reshape(1, D))


def kernel(x, edge_index, Wl0, bl0, Wr0, Wl1, bl1, Wr1, W, b):
    src = edge_index[0].astype(jnp.int32)
    dst = edge_index[1].astype(jnp.int32)
    pkl, nblk = _partition(src, dst)
    a0 = _segmax(x, pkl, nblk, fixup=True)
    h1 = _fused_lin(a0, x, Wl0, bl0, Wr0)
    a1 = _segmax(h1, pkl, nblk, fixup=False)
    out = _final_lin(a1, h1, Wl1, bl1, Wr1, W, b)
    return out


# speedup vs baseline: 2.9721x; 1.1544x over previous
import functools

import jax
import jax.numpy as jnp
from jax import lax
from jax.experimental import pallas as pl
from jax.experimental.pallas import tpu as pltpu
from jax.experimental.pallas import tpu_sc as plsc

N = 10000
E = 320000
D = 128
L = 16
NC = 2
NS = 16
NW = NC * NS
NB = 313
NPAD = NW * NB
K = 512
KSUB = 128
SB = 24
CHUNK = 4000
NCHUNK = E // CHUNK
SORT_CAP = 20480
OVBASE = SORT_CAP + K
CAPR = OVBASE + E + K
OFFP = 320
SHIFT = 14
NEG = float("-inf")

_mesh = lambda: plsc.VectorSubcoreMesh(core_axis_name="c", subcore_axis_name="s")


def _wid():
    return lax.axis_index("s") * NC + lax.axis_index("c")


def _part_body(src_hbm, dst_hbm, pkl, nblk, offsl, srcb0, dstb0, srcb1, dstb1,
               pkb, pkb2, hist, offs, offs_w, tmp, cnt_v, sem0, sem1):
    w = _wid()
    lo = w * NB
    hi = lo + NB
    lanes = lax.iota(jnp.int32, L)

    def _fire(ci, sb, db, sem):
        off = pl.multiple_of(ci * CHUNK, 8)
        pltpu.async_copy(src_hbm.at[pl.ds(off, CHUNK)], sb, sem)
        pltpu.async_copy(dst_hbm.at[pl.ds(off, CHUNK)], db, sem)

    def _wait(ci, sb, db, sem):
        off = pl.multiple_of(ci * CHUNK, 8)
        pltpu.make_async_copy(src_hbm.at[pl.ds(off, CHUNK)], sb, sem).wait()
        pltpu.make_async_copy(dst_hbm.at[pl.ds(off, CHUNK)], db, sem).wait()

    def _scan(sb, db, carry):
        off_v, ov = carry

        def scan_vreg(i, off_v):
            dd = db[pl.ds(i * L, L)]
            ss = sb[pl.ds(i * L, L)]
            m = (dd >= lo) & (dd < hi)
            ones = jnp.where(m, 1, 0).astype(jnp.int32)
            pos = off_v + plsc.cumsum(ones) - 1
            pk = ss | ((dd - lo) << SHIFT)
            plsc.store_scatter(pkb, [pos], pk, mask=m)
            return off_v + plsc.all_reduce_population_count(m)

        off_v = lax.fori_loop(0, CHUNK // L, scan_vreg, off_v)
        off_s = jnp.max(off_v)
        nov = jnp.maximum(off_s - SORT_CAP, 0) // K

        def spill(j, _):
            pltpu.sync_copy(
                pkb.at[pl.ds(SORT_CAP + j * K, K)],
                pkl.at[pl.ds(pl.multiple_of(w * CAPR + OVBASE + (ov + j) * K, 8), K)])
            return 0

        lax.fori_loop(0, nov, spill, 0)
        r = off_s - nov * K
        nmv = jnp.where(nov > 0, (r - SORT_CAP + L - 1) // L, 0)

        def mv(j, _):
            pkb[pl.ds(SORT_CAP + j * L, L)] = plsc.load_gather(
                pkb, [SORT_CAP + nov * K + j * L + lanes])
            return 0

        lax.fori_loop(0, nmv, mv, 0)
        return off_v - nov * K, ov + nov

    _fire(0, srcb0, dstb0, sem0)

    def chunk_pair(i, carry):
        ca = 2 * i
        _wait(ca, srcb0, dstb0, sem0)
        _fire(ca + 1, srcb1, dstb1, sem1)
        carry = _scan(srcb0, dstb0, carry)
        _wait(ca + 1, srcb1, dstb1, sem1)

        @pl.when(ca + 2 < NCHUNK)
        def _():
            _fire(ca + 2, srcb0, dstb0, sem0)

        carry = _scan(srcb1, dstb1, carry)
        return carry

    off0 = jnp.zeros((L,), jnp.int32)
    off_v, ov = lax.fori_loop(0, NCHUNK // 2, chunk_pair, (off0, jnp.int32(0)))
    r = jnp.max(off_v)
    dummy = jnp.full((L,), NB << SHIFT, jnp.int32)
    plsc.store_scatter(pkb, [r + lanes], dummy)
    rp = ((r + L - 1) // L) * L
    nv = rp // L

    def z(i, _):
        hist[pl.ds(i * L, L)] = jnp.zeros((L,), jnp.int32)
        return 0

    lax.fori_loop(0, OFFP // L, z, 0)

    def _runs(ks):
        tmp[pl.ds(0, L)] = jnp.full((L,), -1, jnp.int32)
        tmp[pl.ds(L, L)] = jnp.full((L,), NB + 2, jnp.int32)
        tmp[pl.ds(1, L)] = ks
        prev = tmp[pl.ds(0, L)]
        nxt = tmp[pl.ds(2, L)]
        chg = ks != prev
        endm = ks != nxt
        run_start = plsc.cummax(jnp.where(chg, lanes, 0))
        return run_start, endm

    def h1(j, _):
        pk = pkb[pl.ds(j * L, L)]
        kk = pk >> SHIFT
        ks, pks = plsc.sort_key_val(kk, pk)
        pkb[pl.ds(j * L, L)] = pks
        run_start, endm = _runs(ks)
        rlen = lanes - run_start + 1
        plsc.addupdate_scatter(hist, [ks], rlen, mask=endm)
        return 0

    lax.fori_loop(0, nv, h1, 0)

    def pfx(i, carry):
        h = hist[pl.ds(i * L, L)]
        c = plsc.cumsum(h)
        ex = carry + c - h
        offs[pl.ds(i * L, L)] = ex
        offs_w[pl.ds(i * L, L)] = ex
        return carry + jnp.max(c)

    lax.fori_loop(0, OFFP // L, pfx, jnp.int32(0))

    def p2(j, _):
        pks = pkb[pl.ds(j * L, L)]
        ks = pks >> SHIFT
        run_start, endm = _runs(ks)
        basev = plsc.load_gather(offs, [ks])
        plsc.store_scatter(pkb2, [basev + lanes - run_start], pks)
        plsc.addupdate_scatter(offs, [ks], lanes - run_start + 1, mask=endm)
        return 0

    lax.fori_loop(0, nv, p2, 0)

    npad = ((rp + K - 1) // K) * K
    nsb = npad // K

    def kp(j, _):
        plsc.store_scatter(pkb2, [rp + j * L + lanes], dummy)
        return 0

    lax.fori_loop(0, (npad - rp) // L, kp, 0)
    plsc.store_scatter(offs_w, [jnp.full((L,), NB + 1, jnp.int32)],
                       jnp.zeros((L,), jnp.int32) + npad, mask=lanes == 0)

    def flush(b, _):
        pltpu.sync_copy(pkb2.at[pl.ds(b * K, K)],
                        pkl.at[pl.ds(pl.multiple_of(w * CAPR + b * K, 8), K)])
        return 0

    lax.fori_loop(0, nsb, flush, 0)
    pltpu.sync_copy(offs_w, offsl.at[pl.ds(pl.multiple_of(w * OFFP, 8), OFFP)])
    cnt_v[...] = jnp.zeros((L,), jnp.int32) + (nsb | (ov << 8))
    pltpu.sync_copy(cnt_v, nblk.at[pl.ds(pl.multiple_of(w * L, 8), L)])


def _partition(src, dst):
    return pl.kernel(
        _part_body,
        out_type=[
            jax.ShapeDtypeStruct((NW * CAPR,), jnp.int32),
            jax.ShapeDtypeStruct((NW * L,), jnp.int32),
            jax.ShapeDtypeStruct((NW * OFFP,), jnp.int32),
        ],
        mesh=_mesh(),
        compiler_params=pltpu.CompilerParams(needs_layout_passes=False),
        scratch_types=[
            pltpu.VMEM((CHUNK,), jnp.int32),
            pltpu.VMEM((CHUNK,), jnp.int32),
            pltpu.VMEM((CHUNK,), jnp.int32),
            pltpu.VMEM((CHUNK,), jnp.int32),
            pltpu.VMEM((SORT_CAP + K + CHUNK + L,), jnp.int32),
            pltpu.VMEM((SORT_CAP + K + L,), jnp.int32),
            pltpu.VMEM((OFFP,), jnp.int32),
            pltpu.VMEM((OFFP,), jnp.int32),
            pltpu.VMEM((OFFP,), jnp.int32),
            pltpu.VMEM((3 * L,), jnp.int32),
            pltpu.VMEM((L,), jnp.int32),
            pltpu.SemaphoreType.DMA,
            pltpu.SemaphoreType.DMA,
        ],
    )(src, dst)


def _seg_body(x_hbm, pkl, nblk, offsl, outf, acc1, pk_stage, pk_v, idx_v,
              rows_v, offs_v, cnt_v, sem, *, fixup):
    w = _wid()
    pltpu.sync_copy(nblk.at[pl.ds(pl.multiple_of(w * L, 8), L)], cnt_v)
    both = jnp.max(cnt_v[...])
    nsb = both & 0xFF
    nov = both >> 8
    pltpu.sync_copy(offsl.at[pl.ds(pl.multiple_of(w * OFFP, 8), OFFP)], offs_v)

    init = NEG if fixup else 0.0

    def ini(i, _):
        acc1[pl.ds(i * L, L)] = jnp.full((L,), init, jnp.float32)
        return 0

    lax.fori_loop(0, (NB + 1) * D // L, ini, 0)

    pltpu.sync_copy(pkl.at[pl.ds(pl.multiple_of(w * CAPR, 8), SB * K)], pk_stage)

    lanes = lax.iota(jnp.int32, L)
    zero_v = jnp.zeros((L,), jnp.int32)

    def _gather_rows(pkref, boff):
        def unp(j, _):
            v = pkref[pl.ds(boff + j * L, L)]
            idx_v[pl.ds(j * L, L)] = v & ((1 << SHIFT) - 1)
            return 0

        lax.fori_loop(0, K // L, unp, 0)
        cps = [
            pltpu.async_copy(x_hbm.at[idx_v.at[pl.ds(j * KSUB, KSUB)]],
                             rows_v.at[pl.ds(j * KSUB, KSUB)], sem)
            for j in range(K // KSUB)
        ]
        for cp in cps:
            cp.wait()

    def _process_sorted(pkref, boff, b):
        _gather_rows(pkref, boff)
        l_first = jnp.max(plsc.load_gather(pkref, [boff + zero_v])) >> SHIFT
        l_last = jnp.max(plsc.load_gather(pkref, [boff + K - 1 + zero_v])) >> SHIFT
        blo = b * K
        bhi = blo + K

        def node(l, _):
            s = jnp.max(plsc.load_gather(offs_v, [l + zero_v]))
            e = jnp.max(plsc.load_gather(offs_v, [l + 1 + zero_v]))
            s2 = jnp.maximum(s, blo) - blo
            e2 = jnp.minimum(e, bhi) - blo
            iis = [l * D + f * L + lanes for f in range(D // L)]
            accs = [plsc.load_gather(acc1, [ii]) for ii in iis]

            def ee(k, accs_c):
                rrs = [rows_v[k, pl.ds(f * L, L)] for f in range(D // L)]
                return tuple(jnp.maximum(a, rr) for a, rr in zip(accs_c, rrs))

            accs = lax.fori_loop(s2, e2, ee, tuple(accs))
            for ii, a in zip(iis, accs):
                plsc.store_scatter(acc1, [ii], a)
            return 0

        lax.fori_loop(l_first, l_last + 1, node, 0)
        return 0

    def batch_staged(b, _):
        return _process_sorted(pk_stage, b * K, b)

    lax.fori_loop(0, jnp.minimum(nsb, SB), batch_staged, 0)

    def batch_hbm(b, _):
        pltpu.sync_copy(pkl.at[pl.ds(pl.multiple_of(w * CAPR + b * K, 8), K)],
                        pk_v)
        return _process_sorted(pk_v, 0, b)

    lax.fori_loop(SB, nsb, batch_hbm, 0)

    def batch_ov(b, _):
        pltpu.sync_copy(
            pkl.at[pl.ds(pl.multiple_of(w * CAPR + OVBASE + b * K, 8), K)],
            pk_v)
        _gather_rows(pk_v, 0)

        def edge(k, _):
            lsp = plsc.load_gather(pk_v, [k + zero_v]) >> SHIFT
            base = lsp * D
            iis = [base + f * L + lanes for f in range(D // L)]
            accs = [plsc.load_gather(acc1, [ii]) for ii in iis]
            rrs = [rows_v[k, pl.ds(f * L, L)] for f in range(D // L)]
            for ii, a, rr in zip(iis, accs, rrs):
                plsc.store_scatter(acc1, [ii], jnp.maximum(a, rr))
            return 0

        lax.fori_loop(0, K, edge, 0)
        return 0

    lax.fori_loop(0, nov, batch_ov, 0)

    if fixup:
        def fix(i, _):
            v = acc1[pl.ds(i * L, L)]
            acc1[pl.ds(i * L, L)] = jnp.where(v == NEG, 0.0, v)
            return 0

        lax.fori_loop(0, NB * D // L, fix, 0)

    pltpu.sync_copy(acc1.at[pl.ds(0, NB * D)],
                    outf.at[pl.ds(pl.multiple_of(w * NB * D, 8), NB * D)])


def _segmax(x2d, pkl, nblk, offsl, fixup):
    out = pl.kernel(
        functools.partial(_seg_body, fixup=fixup),
        out_type=jax.ShapeDtypeStruct((NPAD * D,), jnp.float32),
        mesh=_mesh(),
        compiler_params=pltpu.CompilerParams(needs_layout_passes=False),
        scratch_types=[
            pltpu.VMEM(((NB + 1) * D,), jnp.float32),
            pltpu.VMEM((SB * K,), jnp.int32),
            pltpu.VMEM((K,), jnp.int32),
            pltpu.VMEM((K,), jnp.int32),
            pltpu.VMEM((K, D), jnp.float32),
            pltpu.VMEM((OFFP,), jnp.int32),
            pltpu.VMEM((L,), jnp.int32),
            pltpu.SemaphoreType.DMA,
        ],
    )(x2d, pkl, nblk, offsl)
    return out.reshape(NPAD, D)


_ROWS = 1000


def _lin_body(a_ref, x_ref, wl_ref, bl_ref, wr_ref, o_ref):
    acc = lax.dot_general(a_ref[...], wl_ref[...], (((1,), (1,)), ((), ())),
                          preferred_element_type=jnp.float32)
    acc += lax.dot_general(x_ref[...], wr_ref[...], (((1,), (1,)), ((), ())),
                           preferred_element_type=jnp.float32)
    acc += bl_ref[...]
    o_ref[...] = jnp.maximum(acc, 0.0)


def _fused_lin(a, x, wl, bl, wr):
    return pl.pallas_call(
        _lin_body,
        grid=(N // _ROWS,),
        in_specs=[
            pl.BlockSpec((_ROWS, D), lambda i: (i, 0)),
            pl.BlockSpec((_ROWS, D), lambda i: (i, 0)),
            pl.BlockSpec((D, D), lambda i: (0, 0)),
            pl.BlockSpec((1, D), lambda i: (0, 0)),
            pl.BlockSpec((D, D), lambda i: (0, 0)),
        ],
        out_specs=pl.BlockSpec((_ROWS, D), lambda i: (i, 0)),
        out_shape=jax.ShapeDtypeStruct((N, D), jnp.float32),
    )(a, x, wl, bl.reshape(1, D), wr)


def _final_body(a_ref, x_ref, wl_ref, bl_ref, wr_ref, wf_ref, bf_ref, o_ref):
    acc = lax.dot_general(a_ref[...], wl_ref[...], (((1,), (1,)), ((), ())),
                          preferred_element_type=jnp.float32)
    acc += lax.dot_general(x_ref[...], wr_ref[...], (((1,), (1,)), ((), ())),
                           preferred_element_type=jnp.float32)
    h = jnp.maximum(acc + bl_ref[...], 0.0)
    o_ref[...] = lax.dot_general(h, wf_ref[...], (((1,), (1,)), ((), ())),
                                 preferred_element_type=jnp.float32) + bf_ref[...]


def _final_lin(a, x, wl, bl, wr, wf, bf):
    return pl.pallas_call(
        _final_body,
        grid=(N // _ROWS,),
        in_specs=[
            pl.BlockSpec((_ROWS, D), lambda i: (i, 0)),
            pl.BlockSpec((_ROWS, D), lambda i: (i, 0)),
            pl.BlockSpec((D, D), lambda i: (0, 0)),
            pl.BlockSpec((1, D), lambda i: (0, 0)),
            pl.BlockSpec((D, D), lambda i: (0, 0)),
            pl.BlockSpec((D, D), lambda i: (0, 0)),
            pl.BlockSpec((1, D), lambda i: (0, 0)),
        ],
        out_specs=pl.BlockSpec((_ROWS, D), lambda i: (i, 0)),
        out_shape=jax.ShapeDtypeStruct((N, D), jnp.float32),
    )(a, x, wl, bl.reshape(1, D), wr, wf, bf.reshape(1, D))


def kernel(x, edge_index, Wl0, bl0, Wr0, Wl1, bl1, Wr1, W, b):
    src = edge_index[0].astype(jnp.int32)
    dst = edge_index[1].astype(jnp.int32)
    pkl, nblk, offsl = _partition(src, dst)
    a0 = _segmax(x, pkl, nblk, offsl, fixup=True)
    h1 = _fused_lin(a0, x, Wl0, bl0, Wr0)
    a1 = _segmax(h1, pkl, nblk, offsl, fixup=False)
    out = _final_lin(a1, h1, Wl1, bl1, Wr1, W, b)
    return out


# speedup vs baseline: 3.0335x; 1.0207x over previous
import functools

import jax
import jax.numpy as jnp
from jax import lax
from jax.experimental import pallas as pl
from jax.experimental.pallas import tpu as pltpu
from jax.experimental.pallas import tpu_sc as plsc

N = 10000
E = 320000
D = 128
L = 16
NC = 2
NS = 16
NW = NC * NS
NB = 313
NPAD = NW * NB
K = 512
KSUB = 128
SB = 24
CHUNK = 4000
NCHUNK = E // CHUNK
SORT_CAP = 20480
OVBASE = SORT_CAP + K
CAPR = OVBASE + E + K
OFFP = 320
SHIFT = 14
NEG = float("-inf")

_mesh = lambda: plsc.VectorSubcoreMesh(core_axis_name="c", subcore_axis_name="s")


def _wid():
    return lax.axis_index("s") * NC + lax.axis_index("c")


def _part_body(src_hbm, dst_hbm, pkl, nblk, offsl, srcb0, dstb0, srcb1, dstb1,
               pkb, pkb2, hist, offs, offs_w, tmp, cnt_v, sem0, sem1):
    w = _wid()
    lo = w * NB
    hi = lo + NB
    lanes = lax.iota(jnp.int32, L)

    def _fire(ci, sb, db, sem):
        off = pl.multiple_of(ci * CHUNK, 8)
        pltpu.async_copy(src_hbm.at[pl.ds(off, CHUNK)], sb, sem)
        pltpu.async_copy(dst_hbm.at[pl.ds(off, CHUNK)], db, sem)

    def _wait(ci, sb, db, sem):
        off = pl.multiple_of(ci * CHUNK, 8)
        pltpu.make_async_copy(src_hbm.at[pl.ds(off, CHUNK)], sb, sem).wait()
        pltpu.make_async_copy(dst_hbm.at[pl.ds(off, CHUNK)], db, sem).wait()

    def _scan(sb, db, carry):
        off_v, ov = carry

        def scan_vreg(i, off_v):
            dd = db[pl.ds(i * L, L)]
            ss = sb[pl.ds(i * L, L)]
            m = (dd >= lo) & (dd < hi)
            ones = jnp.where(m, 1, 0).astype(jnp.int32)
            pos = off_v + plsc.cumsum(ones) - 1
            pk = ss | ((dd - lo) << SHIFT)
            plsc.store_scatter(pkb, [pos], pk, mask=m)
            return off_v + plsc.all_reduce_population_count(m)

        off_v = lax.fori_loop(0, CHUNK // L, scan_vreg, off_v)
        off_s = jnp.max(off_v)
        nov = jnp.maximum(off_s - SORT_CAP, 0) // K

        def spill(j, _):
            pltpu.sync_copy(
                pkb.at[pl.ds(SORT_CAP + j * K, K)],
                pkl.at[pl.ds(pl.multiple_of(w * CAPR + OVBASE + (ov + j) * K, 8), K)])
            return 0

        lax.fori_loop(0, nov, spill, 0)
        r = off_s - nov * K
        nmv = jnp.where(nov > 0, (r - SORT_CAP + L - 1) // L, 0)

        def mv(j, _):
            pkb[pl.ds(SORT_CAP + j * L, L)] = plsc.load_gather(
                pkb, [SORT_CAP + nov * K + j * L + lanes])
            return 0

        lax.fori_loop(0, nmv, mv, 0)
        return off_v - nov * K, ov + nov

    _fire(0, srcb0, dstb0, sem0)

    def chunk_pair(i, carry):
        ca = 2 * i
        _wait(ca, srcb0, dstb0, sem0)
        _fire(ca + 1, srcb1, dstb1, sem1)
        carry = _scan(srcb0, dstb0, carry)
        _wait(ca + 1, srcb1, dstb1, sem1)

        @pl.when(ca + 2 < NCHUNK)
        def _():
            _fire(ca + 2, srcb0, dstb0, sem0)

        carry = _scan(srcb1, dstb1, carry)
        return carry

    off0 = jnp.zeros((L,), jnp.int32)
    off_v, ov = lax.fori_loop(0, NCHUNK // 2, chunk_pair, (off0, jnp.int32(0)))
    r = jnp.max(off_v)
    dummy = jnp.full((L,), NB << SHIFT, jnp.int32)
    plsc.store_scatter(pkb, [r + lanes], dummy)
    rp = ((r + L - 1) // L) * L
    nv = rp // L

    def z(i, _):
        hist[pl.ds(i * L, L)] = jnp.zeros((L,), jnp.int32)
        return 0

    lax.fori_loop(0, OFFP // L, z, 0)

    def _runs(ks):
        tmp[pl.ds(0, L)] = jnp.full((L,), -1, jnp.int32)
        tmp[pl.ds(L, L)] = jnp.full((L,), NB + 2, jnp.int32)
        tmp[pl.ds(1, L)] = ks
        prev = tmp[pl.ds(0, L)]
        nxt = tmp[pl.ds(2, L)]
        chg = ks != prev
        endm = ks != nxt
        run_start = plsc.cummax(jnp.where(chg, lanes, 0))
        return run_start, endm

    def h1(j, _):
        pk = pkb[pl.ds(j * L, L)]
        kk = pk >> SHIFT
        ks, pks = plsc.sort_key_val(kk, pk)
        pkb[pl.ds(j * L, L)] = pks
        run_start, endm = _runs(ks)
        rlen = lanes - run_start + 1
        plsc.addupdate_scatter(hist, [ks], rlen, mask=endm)
        return 0

    lax.fori_loop(0, nv, h1, 0)

    def pfx(i, carry):
        h = hist[pl.ds(i * L, L)]
        c = plsc.cumsum(h)
        ex = carry + c - h
        offs[pl.ds(i * L, L)] = ex
        offs_w[pl.ds(i * L, L)] = ex
        return carry + jnp.max(c)

    lax.fori_loop(0, OFFP // L, pfx, jnp.int32(0))

    def p2(j, _):
        pks = pkb[pl.ds(j * L, L)]
        ks = pks >> SHIFT
        run_start, endm = _runs(ks)
        basev = plsc.load_gather(offs, [ks])
        plsc.store_scatter(pkb2, [basev + lanes - run_start], pks)
        plsc.addupdate_scatter(offs, [ks], lanes - run_start + 1, mask=endm)
        return 0

    lax.fori_loop(0, nv, p2, 0)

    npad = ((rp + K - 1) // K) * K
    nsb = npad // K

    def kp(j, _):
        plsc.store_scatter(pkb2, [rp + j * L + lanes], dummy)
        return 0

    lax.fori_loop(0, (npad - rp) // L, kp, 0)
    plsc.store_scatter(offs_w, [jnp.full((L,), NB + 1, jnp.int32)],
                       jnp.zeros((L,), jnp.int32) + npad, mask=lanes == 0)

    def flush(b, _):
        pltpu.sync_copy(pkb2.at[pl.ds(b * K, K)],
                        pkl.at[pl.ds(pl.multiple_of(w * CAPR + b * K, 8), K)])
        return 0

    lax.fori_loop(0, nsb, flush, 0)
    pltpu.sync_copy(offs_w, offsl.at[pl.ds(pl.multiple_of(w * OFFP, 8), OFFP)])
    cnt_v[...] = jnp.zeros((L,), jnp.int32) + (nsb | (ov << 8))
    pltpu.sync_copy(cnt_v, nblk.at[pl.ds(pl.multiple_of(w * L, 8), L)])


def _partition(src, dst):
    return pl.kernel(
        _part_body,
        out_type=[
            jax.ShapeDtypeStruct((NW * CAPR,), jnp.int32),
            jax.ShapeDtypeStruct((NW * L,), jnp.int32),
            jax.ShapeDtypeStruct((NW * OFFP,), jnp.int32),
        ],
        mesh=_mesh(),
        compiler_params=pltpu.CompilerParams(needs_layout_passes=False),
        scratch_types=[
            pltpu.VMEM((CHUNK,), jnp.int32),
            pltpu.VMEM((CHUNK,), jnp.int32),
            pltpu.VMEM((CHUNK,), jnp.int32),
            pltpu.VMEM((CHUNK,), jnp.int32),
            pltpu.VMEM((SORT_CAP + K + CHUNK + L,), jnp.int32),
            pltpu.VMEM((SORT_CAP + K + L,), jnp.int32),
            pltpu.VMEM((OFFP,), jnp.int32),
            pltpu.VMEM((OFFP,), jnp.int32),
            pltpu.VMEM((OFFP,), jnp.int32),
            pltpu.VMEM((3 * L,), jnp.int32),
            pltpu.VMEM((L,), jnp.int32),
            pltpu.SemaphoreType.DMA,
            pltpu.SemaphoreType.DMA,
        ],
    )(src, dst)


def _seg_body(x_hbm, pkl, nblk, offsl, outf, acc1, pk_stage, pk_v, idx_v,
              rows_v, offs_v, cnt_v, sem, *, fixup):
    w = _wid()
    pltpu.sync_copy(nblk.at[pl.ds(pl.multiple_of(w * L, 8), L)], cnt_v)
    both = jnp.max(cnt_v[...])
    nsb = both & 0xFF
    nov = both >> 8
    pltpu.sync_copy(offsl.at[pl.ds(pl.multiple_of(w * OFFP, 8), OFFP)], offs_v)

    init = NEG if fixup else 0.0

    @plsc.parallel_loop(0, (NB + 1) * D // L, unroll=4)
    def ini(i):
        acc1[pl.ds(i * L, L)] = jnp.full((L,), init, jnp.float32)

    pltpu.sync_copy(pkl.at[pl.ds(pl.multiple_of(w * CAPR, 8), SB * K)], pk_stage)

    lanes = lax.iota(jnp.int32, L)
    zero_v = jnp.zeros((L,), jnp.int32)

    def _gather_rows(pkref, boff):
        @plsc.parallel_loop(0, K // L, unroll=4)
        def unp(j):
            v = pkref[pl.ds(boff + j * L, L)]
            idx_v[pl.ds(j * L, L)] = v & ((1 << SHIFT) - 1)
        cps = [
            pltpu.async_copy(x_hbm.at[idx_v.at[pl.ds(j * KSUB, KSUB)]],
                             rows_v.at[pl.ds(j * KSUB, KSUB)], sem)
            for j in range(K // KSUB)
        ]
        for cp in cps:
            cp.wait()

    def _process_sorted(pkref, boff, b):
        _gather_rows(pkref, boff)
        l_first = jnp.max(plsc.load_gather(pkref, [boff + zero_v])) >> SHIFT
        l_last = jnp.max(plsc.load_gather(pkref, [boff + K - 1 + zero_v])) >> SHIFT
        blo = b * K
        bhi = blo + K

        def node(l, _):
            s = jnp.max(plsc.load_gather(offs_v, [l + zero_v]))
            e = jnp.max(plsc.load_gather(offs_v, [l + 1 + zero_v]))
            s2 = jnp.maximum(s, blo) - blo
            e2 = jnp.minimum(e, bhi) - blo
            iis = [l * D + f * L + lanes for f in range(D // L)]
            accs = [plsc.load_gather(acc1, [ii]) for ii in iis]

            def ee(k, accs_c):
                rrs = [rows_v[k, pl.ds(f * L, L)] for f in range(D // L)]
                return tuple(jnp.maximum(a, rr) for a, rr in zip(accs_c, rrs))

            accs = plsc.parallel_loop(s2, e2, unroll=2, carry=tuple(accs))(ee)
            for ii, a in zip(iis, accs):
                plsc.store_scatter(acc1, [ii], a)
            return 0

        lax.fori_loop(l_first, l_last + 1, node, 0)
        return 0

    def batch_staged(b, _):
        return _process_sorted(pk_stage, b * K, b)

    lax.fori_loop(0, jnp.minimum(nsb, SB), batch_staged, 0)

    def batch_hbm(b, _):
        pltpu.sync_copy(pkl.at[pl.ds(pl.multiple_of(w * CAPR + b * K, 8), K)],
                        pk_v)
        return _process_sorted(pk_v, 0, b)

    lax.fori_loop(SB, nsb, batch_hbm, 0)

    def batch_ov(b, _):
        pltpu.sync_copy(
            pkl.at[pl.ds(pl.multiple_of(w * CAPR + OVBASE + b * K, 8), K)],
            pk_v)
        _gather_rows(pk_v, 0)

        def edge(k, _):
            lsp = plsc.load_gather(pk_v, [k + zero_v]) >> SHIFT
            base = lsp * D
            iis = [base + f * L + lanes for f in range(D // L)]
            accs = [plsc.load_gather(acc1, [ii]) for ii in iis]
            rrs = [rows_v[k, pl.ds(f * L, L)] for f in range(D // L)]
            for ii, a, rr in zip(iis, accs, rrs):
                plsc.store_scatter(acc1, [ii], jnp.maximum(a, rr))
            return 0

        lax.fori_loop(0, K, edge, 0)
        return 0

    lax.fori_loop(0, nov, batch_ov, 0)

    if fixup:
        @plsc.parallel_loop(0, NB * D // L, unroll=4)
        def fix(i):
            v = acc1[pl.ds(i * L, L)]
            acc1[pl.ds(i * L, L)] = jnp.where(v == NEG, 0.0, v)

    pltpu.sync_copy(acc1.at[pl.ds(0, NB * D)],
                    outf.at[pl.ds(pl.multiple_of(w * NB * D, 8), NB * D)])


def _segmax(x2d, pkl, nblk, offsl, fixup):
    out = pl.kernel(
        functools.partial(_seg_body, fixup=fixup),
        out_type=jax.ShapeDtypeStruct((NPAD * D,), jnp.float32),
        mesh=_mesh(),
        compiler_params=pltpu.CompilerParams(needs_layout_passes=False),
        scratch_types=[
            pltpu.VMEM(((NB + 1) * D,), jnp.float32),
            pltpu.VMEM((SB * K,), jnp.int32),
            pltpu.VMEM((K,), jnp.int32),
            pltpu.VMEM((K,), jnp.int32),
            pltpu.VMEM((K, D), jnp.float32),
            pltpu.VMEM((OFFP,), jnp.int32),
            pltpu.VMEM((L,), jnp.int32),
            pltpu.SemaphoreType.DMA,
        ],
    )(x2d, pkl, nblk, offsl)
    return out.reshape(NPAD, D)


_ROWS = 1000


def _lin_body(a_ref, x_ref, wl_ref, bl_ref, wr_ref, o_ref):
    acc = lax.dot_general(a_ref[...], wl_ref[...], (((1,), (1,)), ((), ())),
                          preferred_element_type=jnp.float32)
    acc += lax.dot_general(x_ref[...], wr_ref[...], (((1,), (1,)), ((), ())),
                           preferred_element_type=jnp.float32)
    acc += bl_ref[...]
    o_ref[...] = jnp.maximum(acc, 0.0)


def _fused_lin(a, x, wl, bl, wr):
    return pl.pallas_call(
        _lin_body,
        grid=(N // _ROWS,),
        in_specs=[
            pl.BlockSpec((_ROWS, D), lambda i: (i, 0)),
            pl.BlockSpec((_ROWS, D), lambda i: (i, 0)),
            pl.BlockSpec((D, D), lambda i: (0, 0)),
            pl.BlockSpec((1, D), lambda i: (0, 0)),
            pl.BlockSpec((D, D), lambda i: (0, 0)),
        ],
        out_specs=pl.BlockSpec((_ROWS, D), lambda i: (i, 0)),
        out_shape=jax.ShapeDtypeStruct((N, D), jnp.float32),
    )(a, x, wl, bl.reshape(1, D), wr)


def _final_body(a_ref, x_ref, wl_ref, bl_ref, wr_ref, wf_ref, bf_ref, o_ref):
    acc = lax.dot_general(a_ref[...], wl_ref[...], (((1,), (1,)), ((), ())),
                          preferred_element_type=jnp.float32)
    acc += lax.dot_general(x_ref[...], wr_ref[...], (((1,), (1,)), ((), ())),
                           preferred_element_type=jnp.float32)
    h = jnp.maximum(acc + bl_ref[...], 0.0)
    o_ref[...] = lax.dot_general(h, wf_ref[...], (((1,), (1,)), ((), ())),
                                 preferred_element_type=jnp.float32) + bf_ref[...]


def _final_lin(a, x, wl, bl, wr, wf, bf):
    return pl.pallas_call(
        _final_body,
        grid=(N // _ROWS,),
        in_specs=[
            pl.BlockSpec((_ROWS, D), lambda i: (i, 0)),
            pl.BlockSpec((_ROWS, D), lambda i: (i, 0)),
            pl.BlockSpec((D, D), lambda i: (0, 0)),
            pl.BlockSpec((1, D), lambda i: (0, 0)),
            pl.BlockSpec((D, D), lambda i: (0, 0)),
            pl.BlockSpec((D, D), lambda i: (0, 0)),
            pl.BlockSpec((1, D), lambda i: (0, 0)),
        ],
        out_specs=pl.BlockSpec((_ROWS, D), lambda i: (i, 0)),
        out_shape=jax.ShapeDtypeStruct((N, D), jnp.float32),
    )(a, x, wl, bl.reshape(1, D), wr, wf, bf.reshape(1, D))


def kernel(x, edge_index, Wl0, bl0, Wr0, Wl1, bl1, Wr1, W, b):
    src = edge_index[0].astype(jnp.int32)
    dst = edge_index[1].astype(jnp.int32)
    pkl, nblk, offsl = _partition(src, dst)
    a0 = _segmax(x, pkl, nblk, offsl, fixup=True)
    h1 = _fused_lin(a0, x, Wl0, bl0, Wr0)
    a1 = _segmax(h1, pkl, nblk, offsl, fixup=False)
    out = _final_lin(a1, h1, Wl1, bl1, Wr1, W, b)
    return out


# speedup vs baseline: 4.3854x; 1.4456x over previous
import functools

import jax
import jax.numpy as jnp
from jax import lax
from jax.experimental import pallas as pl
from jax.experimental.pallas import tpu as pltpu
from jax.experimental.pallas import tpu_sc as plsc

N = 10000
E = 320000
D = 128
L = 16
NC = 2
NS = 16
NW = NC * NS
NB = 313
NPAD = NW * NB
K = 256
KSUB = 128
SB = 40
CHUNK = 4000
NCHUNK = E // CHUNK
SORT_CAP = 20480
OVBASE = SORT_CAP + K
CAPR = OVBASE + E + K
OFFP = 320
SHIFT = 14
NEG = float("-inf")

_mesh = lambda: plsc.VectorSubcoreMesh(core_axis_name="c", subcore_axis_name="s")


def _wid():
    return lax.axis_index("s") * NC + lax.axis_index("c")


def _part_body(src_hbm, dst_hbm, pkl, nblk, offsl, srcb0, dstb0, srcb1, dstb1,
               pkb, pkb2, hist, offs, offs_w, tmp, cnt_v, sem0, sem1):
    w = _wid()
    lo = w * NB
    hi = lo + NB
    lanes = lax.iota(jnp.int32, L)

    def _fire(ci, sb, db, sem):
        off = pl.multiple_of(ci * CHUNK, 8)
        pltpu.async_copy(src_hbm.at[pl.ds(off, CHUNK)], sb, sem)
        pltpu.async_copy(dst_hbm.at[pl.ds(off, CHUNK)], db, sem)

    def _wait(ci, sb, db, sem):
        off = pl.multiple_of(ci * CHUNK, 8)
        pltpu.make_async_copy(src_hbm.at[pl.ds(off, CHUNK)], sb, sem).wait()
        pltpu.make_async_copy(dst_hbm.at[pl.ds(off, CHUNK)], db, sem).wait()

    def _scan(sb, db, carry):
        off_v, ov = carry

        def scan_vreg(i, off_v):
            dd = db[pl.ds(i * L, L)]
            ss = sb[pl.ds(i * L, L)]
            m = (dd >= lo) & (dd < hi)
            ones = jnp.where(m, 1, 0).astype(jnp.int32)
            pos = off_v + plsc.cumsum(ones) - 1
            pk = ss | ((dd - lo) << SHIFT)
            plsc.store_scatter(pkb, [pos], pk, mask=m)
            return off_v + plsc.all_reduce_population_count(m)

        off_v = lax.fori_loop(0, CHUNK // L, scan_vreg, off_v)
        off_s = jnp.max(off_v)
        nov = jnp.maximum(off_s - SORT_CAP, 0) // K

        def spill(j, _):
            pltpu.sync_copy(
                pkb.at[pl.ds(SORT_CAP + j * K, K)],
                pkl.at[pl.ds(pl.multiple_of(w * CAPR + OVBASE + (ov + j) * K, 8), K)])
            return 0

        lax.fori_loop(0, nov, spill, 0)
        r = off_s - nov * K
        nmv = jnp.where(nov > 0, (r - SORT_CAP + L - 1) // L, 0)

        def mv(j, _):
            pkb[pl.ds(SORT_CAP + j * L, L)] = plsc.load_gather(
                pkb, [SORT_CAP + nov * K + j * L + lanes])
            return 0

        lax.fori_loop(0, nmv, mv, 0)
        return off_v - nov * K, ov + nov

    _fire(0, srcb0, dstb0, sem0)

    def chunk_pair(i, carry):
        ca = 2 * i
        _wait(ca, srcb0, dstb0, sem0)
        _fire(ca + 1, srcb1, dstb1, sem1)
        carry = _scan(srcb0, dstb0, carry)
        _wait(ca + 1, srcb1, dstb1, sem1)

        @pl.when(ca + 2 < NCHUNK)
        def _():
            _fire(ca + 2, srcb0, dstb0, sem0)

        carry = _scan(srcb1, dstb1, carry)
        return carry

    off0 = jnp.zeros((L,), jnp.int32)
    off_v, ov = lax.fori_loop(0, NCHUNK // 2, chunk_pair, (off0, jnp.int32(0)))
    r = jnp.max(off_v)
    dummy = jnp.full((L,), NB << SHIFT, jnp.int32)
    plsc.store_scatter(pkb, [r + lanes], dummy)
    rp = ((r + L - 1) // L) * L
    nv = rp // L

    def z(i, _):
        hist[pl.ds(i * L, L)] = jnp.zeros((L,), jnp.int32)
        return 0

    lax.fori_loop(0, OFFP // L, z, 0)

    def _runs(ks):
        tmp[pl.ds(0, L)] = jnp.full((L,), -1, jnp.int32)
        tmp[pl.ds(L, L)] = jnp.full((L,), NB + 2, jnp.int32)
        tmp[pl.ds(1, L)] = ks
        prev = tmp[pl.ds(0, L)]
        nxt = tmp[pl.ds(2, L)]
        chg = ks != prev
        endm = ks != nxt
        run_start = plsc.cummax(jnp.where(chg, lanes, 0))
        return run_start, endm

    def h1(j, _):
        pk = pkb[pl.ds(j * L, L)]
        kk = pk >> SHIFT
        ks, pks = plsc.sort_key_val(kk, pk)
        pkb[pl.ds(j * L, L)] = pks
        run_start, endm = _runs(ks)
        rlen = lanes - run_start + 1
        plsc.addupdate_scatter(hist, [ks], rlen, mask=endm)
        return 0

    lax.fori_loop(0, nv, h1, 0)

    def pfx(i, carry):
        h = hist[pl.ds(i * L, L)]
        c = plsc.cumsum(h)
        ex = carry + c - h
        offs[pl.ds(i * L, L)] = ex
        offs_w[pl.ds(i * L, L)] = ex
        return carry + jnp.max(c)

    lax.fori_loop(0, OFFP // L, pfx, jnp.int32(0))

    def p2(j, _):
        pks = pkb[pl.ds(j * L, L)]
        ks = pks >> SHIFT
        run_start, endm = _runs(ks)
        basev = plsc.load_gather(offs, [ks])
        plsc.store_scatter(pkb2, [basev + lanes - run_start], pks)
        plsc.addupdate_scatter(offs, [ks], lanes - run_start + 1, mask=endm)
        return 0

    lax.fori_loop(0, nv, p2, 0)

    npad = ((rp + K - 1) // K) * K
    nsb = npad // K

    def kp(j, _):
        plsc.store_scatter(pkb2, [rp + j * L + lanes], dummy)
        return 0

    lax.fori_loop(0, (npad - rp) // L, kp, 0)
    plsc.store_scatter(offs_w, [jnp.full((L,), NB + 1, jnp.int32)],
                       jnp.zeros((L,), jnp.int32) + npad, mask=lanes == 0)

    def flush(b, _):
        pltpu.sync_copy(pkb2.at[pl.ds(b * K, K)],
                        pkl.at[pl.ds(pl.multiple_of(w * CAPR + b * K, 8), K)])
        return 0

    lax.fori_loop(0, nsb, flush, 0)
    pltpu.sync_copy(offs_w, offsl.at[pl.ds(pl.multiple_of(w * OFFP, 8), OFFP)])
    cnt_v[...] = jnp.zeros((L,), jnp.int32) + (nsb | (ov << 8))
    pltpu.sync_copy(cnt_v, nblk.at[pl.ds(pl.multiple_of(w * L, 8), L)])


def _partition(src, dst):
    return pl.kernel(
        _part_body,
        out_type=[
            jax.ShapeDtypeStruct((NW * CAPR,), jnp.int32),
            jax.ShapeDtypeStruct((NW * L,), jnp.int32),
            jax.ShapeDtypeStruct((NW * OFFP,), jnp.int32),
        ],
        mesh=_mesh(),
        compiler_params=pltpu.CompilerParams(needs_layout_passes=False),
        scratch_types=[
            pltpu.VMEM((CHUNK,), jnp.int32),
            pltpu.VMEM((CHUNK,), jnp.int32),
            pltpu.VMEM((CHUNK,), jnp.int32),
            pltpu.VMEM((CHUNK,), jnp.int32),
            pltpu.VMEM((SORT_CAP + K + CHUNK + L,), jnp.int32),
            pltpu.VMEM((SORT_CAP + K + L,), jnp.int32),
            pltpu.VMEM((OFFP,), jnp.int32),
            pltpu.VMEM((OFFP,), jnp.int32),
            pltpu.VMEM((OFFP,), jnp.int32),
            pltpu.VMEM((3 * L,), jnp.int32),
            pltpu.VMEM((L,), jnp.int32),
            pltpu.SemaphoreType.DMA,
            pltpu.SemaphoreType.DMA,
        ],
    )(src, dst)


def _seg_body(x_hbm, pkl, nblk, offsl, outf, acc1, pk_stage, pk_v, idx_a,
              idx_b, rows_a, rows_b, offs_v, cnt_v, sem_a, sem_b, *, fixup):
    w = _wid()
    pltpu.sync_copy(nblk.at[pl.ds(pl.multiple_of(w * L, 8), L)], cnt_v)
    both = jnp.max(cnt_v[...])
    nsb = both & 0xFF
    nov = both >> 8
    pltpu.sync_copy(offsl.at[pl.ds(pl.multiple_of(w * OFFP, 8), OFFP)], offs_v)

    init = NEG if fixup else 0.0

    @plsc.parallel_loop(0, (NB + 1) * D // L, unroll=4)
    def ini(i):
        acc1[pl.ds(i * L, L)] = jnp.full((L,), init, jnp.float32)

    pltpu.sync_copy(pkl.at[pl.ds(pl.multiple_of(w * CAPR, 8), SB * K)], pk_stage)

    lanes = lax.iota(jnp.int32, L)
    zero_v = jnp.zeros((L,), jnp.int32)

    def _fire_rows(pkref, boff, idx_v, rows_v, sem):
        @plsc.parallel_loop(0, K // L, unroll=4)
        def unp(j):
            v = pkref[pl.ds(boff + j * L, L)]
            idx_v[pl.ds(j * L, L)] = v & ((1 << SHIFT) - 1)

        for j in range(K // KSUB):
            pltpu.async_copy(x_hbm.at[idx_v.at[pl.ds(j * KSUB, KSUB)]],
                             rows_v.at[pl.ds(j * KSUB, KSUB)], sem)

    def _wait_rows(idx_v, rows_v, sem):
        for j in range(K // KSUB):
            pltpu.make_async_copy(x_hbm.at[idx_v.at[pl.ds(j * KSUB, KSUB)]],
                                  rows_v.at[pl.ds(j * KSUB, KSUB)], sem).wait()

    def _drain_sorted(pkref, boff, rows_v, b):
        l_first = jnp.max(plsc.load_gather(pkref, [boff + zero_v])) >> SHIFT
        l_last = jnp.max(plsc.load_gather(pkref, [boff + K - 1 + zero_v])) >> SHIFT
        blo = b * K
        bhi = blo + K

        def node(l, _):
            s = jnp.max(plsc.load_gather(offs_v, [l + zero_v]))
            e = jnp.max(plsc.load_gather(offs_v, [l + 1 + zero_v]))
            s2 = jnp.maximum(s, blo) - blo
            e2 = jnp.minimum(e, bhi) - blo
            iis = [l * D + f * L + lanes for f in range(D // L)]
            accs = [plsc.load_gather(acc1, [ii]) for ii in iis]

            def ee(k, accs_c):
                rrs = [rows_v[k, pl.ds(f * L, L)] for f in range(D // L)]
                return tuple(jnp.maximum(a, rr) for a, rr in zip(accs_c, rrs))

            accs = plsc.parallel_loop(s2, e2, unroll=2, carry=tuple(accs))(ee)
            for ii, a in zip(iis, accs):
                plsc.store_scatter(acc1, [ii], a)
            return 0

        lax.fori_loop(l_first, l_last + 1, node, 0)

    nsb2 = jnp.minimum(nsb, SB)

    @pl.when(nsb2 > 0)
    def _():
        _fire_rows(pk_stage, 0, idx_a, rows_a, sem_a)

    def pair(p, _):
        ba = 2 * p
        bb = 2 * p + 1
        _wait_rows(idx_a, rows_a, sem_a)

        @pl.when(bb < nsb2)
        def _():
            _fire_rows(pk_stage, bb * K, idx_b, rows_b, sem_b)

        _drain_sorted(pk_stage, ba * K, rows_a, ba)

        @pl.when(bb < nsb2)
        def _():
            _wait_rows(idx_b, rows_b, sem_b)

            @pl.when(bb + 1 < nsb2)
            def _():
                _fire_rows(pk_stage, (bb + 1) * K, idx_a, rows_a, sem_a)

            _drain_sorted(pk_stage, bb * K, rows_b, bb)

        return 0

    lax.fori_loop(0, (nsb2 + 1) // 2, pair, 0)

    def batch_hbm(b, _):
        pltpu.sync_copy(pkl.at[pl.ds(pl.multiple_of(w * CAPR + b * K, 8), K)],
                        pk_v)
        _fire_rows(pk_v, 0, idx_a, rows_a, sem_a)
        _wait_rows(idx_a, rows_a, sem_a)
        _drain_sorted(pk_v, 0, rows_a, b)
        return 0

    lax.fori_loop(SB, nsb, batch_hbm, 0)

    def batch_ov(b, _):
        pltpu.sync_copy(
            pkl.at[pl.ds(pl.multiple_of(w * CAPR + OVBASE + b * K, 8), K)],
            pk_v)
        _fire_rows(pk_v, 0, idx_a, rows_a, sem_a)
        _wait_rows(idx_a, rows_a, sem_a)

        def edge(k, _):
            lsp = plsc.load_gather(pk_v, [k + zero_v]) >> SHIFT
            base = lsp * D
            iis = [base + f * L + lanes for f in range(D // L)]
            accs = [plsc.load_gather(acc1, [ii]) for ii in iis]
            rrs = [rows_a[k, pl.ds(f * L, L)] for f in range(D // L)]
            for ii, a, rr in zip(iis, accs, rrs):
                plsc.store_scatter(acc1, [ii], jnp.maximum(a, rr))
            return 0

        lax.fori_loop(0, K, edge, 0)
        return 0

    lax.fori_loop(0, nov, batch_ov, 0)

    if fixup:
        @plsc.parallel_loop(0, NB * D // L, unroll=4)
        def fix(i):
            v = acc1[pl.ds(i * L, L)]
            acc1[pl.ds(i * L, L)] = jnp.where(v == NEG, 0.0, v)

    pltpu.sync_copy(acc1.at[pl.ds(0, NB * D)],
                    outf.at[pl.ds(pl.multiple_of(w * NB * D, 8), NB * D)])


def _segmax(x2d, pkl, nblk, offsl, fixup):
    out = pl.kernel(
        functools.partial(_seg_body, fixup=fixup),
        out_type=jax.ShapeDtypeStruct((NPAD * D,), jnp.float32),
        mesh=_mesh(),
        compiler_params=pltpu.CompilerParams(needs_layout_passes=False),
        scratch_types=[
            pltpu.VMEM(((NB + 1) * D,), jnp.float32),
            pltpu.VMEM((SB * K,), jnp.int32),
            pltpu.VMEM((K,), jnp.int32),
            pltpu.VMEM((K,), jnp.int32),
            pltpu.VMEM((K,), jnp.int32),
            pltpu.VMEM((K, D), jnp.float32),
            pltpu.VMEM((K, D), jnp.float32),
            pltpu.VMEM((OFFP,), jnp.int32),
            pltpu.VMEM((L,), jnp.int32),
            pltpu.SemaphoreType.DMA,
            pltpu.SemaphoreType.DMA,
        ],
    )(x2d, pkl, nblk, offsl)
    return out.reshape(NPAD, D)


_ROWS = 1000


def _lin_body(a_ref, x_ref, wl_ref, bl_ref, wr_ref, o_ref):
    acc = lax.dot_general(a_ref[...], wl_ref[...], (((1,), (1,)), ((), ())),
                          preferred_element_type=jnp.float32)
    acc += lax.dot_general(x_ref[...], wr_ref[...], (((1,), (1,)), ((), ())),
                           preferred_element_type=jnp.float32)
    acc += bl_ref[...]
    o_ref[...] = jnp.maximum(acc, 0.0)


def _fused_lin(a, x, wl, bl, wr):
    return pl.pallas_call(
        _lin_body,
        grid=(N // _ROWS,),
        in_specs=[
            pl.BlockSpec((_ROWS, D), lambda i: (i, 0)),
            pl.BlockSpec((_ROWS, D), lambda i: (i, 0)),
            pl.BlockSpec((D, D), lambda i: (0, 0)),
            pl.BlockSpec((1, D), lambda i: (0, 0)),
            pl.BlockSpec((D, D), lambda i: (0, 0)),
        ],
        out_specs=pl.BlockSpec((_ROWS, D), lambda i: (i, 0)),
        out_shape=jax.ShapeDtypeStruct((N, D), jnp.float32),
    )(a, x, wl, bl.reshape(1, D), wr)


def _final_body(a_ref, x_ref, wl_ref, bl_ref, wr_ref, wf_ref, bf_ref, o_ref):
    acc = lax.dot_general(a_ref[...], wl_ref[...], (((1,), (1,)), ((), ())),
                          preferred_element_type=jnp.float32)
    acc += lax.dot_general(x_ref[...], wr_ref[...], (((1,), (1,)), ((), ())),
                           preferred_element_type=jnp.float32)
    h = jnp.maximum(acc + bl_ref[...], 0.0)
    o_ref[...] = lax.dot_general(h, wf_ref[...], (((1,), (1,)), ((), ())),
                                 preferred_element_type=jnp.float32) + bf_ref[...]


def _final_lin(a, x, wl, bl, wr, wf, bf):
    return pl.pallas_call(
        _final_body,
        grid=(N // _ROWS,),
        in_specs=[
            pl.BlockSpec((_ROWS, D), lambda i: (i, 0)),
            pl.BlockSpec((_ROWS, D), lambda i: (i, 0)),
            pl.BlockSpec((D, D), lambda i: (0, 0)),
            pl.BlockSpec((1, D), lambda i: (0, 0)),
            pl.BlockSpec((D, D), lambda i: (0, 0)),
            pl.BlockSpec((D, D), lambda i: (0, 0)),
            pl.BlockSpec((1, D), lambda i: (0, 0)),
        ],
        out_specs=pl.BlockSpec((_ROWS, D), lambda i: (i, 0)),
        out_shape=jax.ShapeDtypeStruct((N, D), jnp.float32),
    )(a, x, wl, bl.reshape(1, D), wr, wf, bf.reshape(1, D))


def kernel(x, edge_index, Wl0, bl0, Wr0, Wl1, bl1, Wr1, W, b):
    src = edge_index[0].astype(jnp.int32)
    dst = edge_index[1].astype(jnp.int32)
    pkl, nblk, offsl = _partition(src, dst)
    a0 = _segmax(x, pkl, nblk, offsl, fixup=True)
    h1 = _fused_lin(a0, x, Wl0, bl0, Wr0)
    a1 = _segmax(h1, pkl, nblk, offsl, fixup=False)
    out = _final_lin(a1, h1, Wl1, bl1, Wr1, W, b)
    return out


# speedup vs baseline: 5.5531x; 1.2663x over previous
import functools

import jax
import jax.numpy as jnp
from jax import lax
from jax.experimental import pallas as pl
from jax.experimental.pallas import tpu as pltpu
from jax.experimental.pallas import tpu_sc as plsc

N = 10000
E = 320000
D = 128
L = 16
NC = 2
NS = 16
NW = NC * NS
NB = 313
NPAD = NW * NB
K = 256
KSUB = 128
SB = 40
CHUNK = 4000
NCHUNK = E // CHUNK
SORT_CAP = 20480
OVBASE = SORT_CAP + K
CAPR = OVBASE + E + K
OFFP = 320
SHIFT = 14
NEG = float("-inf")

_mesh = lambda: plsc.VectorSubcoreMesh(core_axis_name="c", subcore_axis_name="s")


def _wid():
    return lax.axis_index("s") * NC + lax.axis_index("c")


def _part_body(src_hbm, dst_hbm, pkl, nblk, offsl, srcb0, dstb0, srcb1, dstb1,
               pkb, pkb2, hist, offs, offs_w, tmp, cnt_v, sem0, sem1):
    w = _wid()
    lo = w * NB
    hi = lo + NB
    lanes = lax.iota(jnp.int32, L)

    def _fire(ci, sb, db, sem):
        off = pl.multiple_of(ci * CHUNK, 8)
        pltpu.async_copy(src_hbm.at[pl.ds(off, CHUNK)], sb, sem)
        pltpu.async_copy(dst_hbm.at[pl.ds(off, CHUNK)], db, sem)

    def _wait(ci, sb, db, sem):
        off = pl.multiple_of(ci * CHUNK, 8)
        pltpu.make_async_copy(src_hbm.at[pl.ds(off, CHUNK)], sb, sem).wait()
        pltpu.make_async_copy(dst_hbm.at[pl.ds(off, CHUNK)], db, sem).wait()

    def _scan(sb, db, carry):
        off_v, ov = carry

        def scan_vreg(i, off_v):
            dd = db[pl.ds(i * L, L)]
            ss = sb[pl.ds(i * L, L)]
            m = (dd >= lo) & (dd < hi)
            ones = jnp.where(m, 1, 0).astype(jnp.int32)
            pos = off_v + plsc.cumsum(ones) - 1
            pk = ss | ((dd - lo) << SHIFT)
            plsc.store_scatter(pkb, [pos], pk, mask=m)
            return off_v + plsc.all_reduce_population_count(m)

        off_v = plsc.parallel_loop(0, CHUNK // L, unroll=2, carry=off_v)(scan_vreg)
        off_s = jnp.max(off_v)
        nov = jnp.maximum(off_s - SORT_CAP, 0) // K

        def spill(j, _):
            pltpu.sync_copy(
                pkb.at[pl.ds(SORT_CAP + j * K, K)],
                pkl.at[pl.ds(pl.multiple_of(w * CAPR + OVBASE + (ov + j) * K, 8), K)])
            return 0

        lax.fori_loop(0, nov, spill, 0)
        r = off_s - nov * K
        nmv = jnp.where(nov > 0, (r - SORT_CAP + L - 1) // L, 0)

        def mv(j, _):
            pkb[pl.ds(SORT_CAP + j * L, L)] = plsc.load_gather(
                pkb, [SORT_CAP + nov * K + j * L + lanes])
            return 0

        lax.fori_loop(0, nmv, mv, 0)
        return off_v - nov * K, ov + nov

    _fire(0, srcb0, dstb0, sem0)

    def chunk_pair(i, carry):
        ca = 2 * i
        _wait(ca, srcb0, dstb0, sem0)
        _fire(ca + 1, srcb1, dstb1, sem1)
        carry = _scan(srcb0, dstb0, carry)
        _wait(ca + 1, srcb1, dstb1, sem1)

        @pl.when(ca + 2 < NCHUNK)
        def _():
            _fire(ca + 2, srcb0, dstb0, sem0)

        carry = _scan(srcb1, dstb1, carry)
        return carry

    off0 = jnp.zeros((L,), jnp.int32)
    off_v, ov = lax.fori_loop(0, NCHUNK // 2, chunk_pair, (off0, jnp.int32(0)))
    r = jnp.max(off_v)
    dummy = jnp.full((L,), NB << SHIFT, jnp.int32)
    plsc.store_scatter(pkb, [r + lanes], dummy)
    rp = ((r + L - 1) // L) * L
    nv = rp // L

    def z(i, _):
        hist[pl.ds(i * L, L)] = jnp.zeros((L,), jnp.int32)
        return 0

    lax.fori_loop(0, OFFP // L, z, 0)

    def _runs(ks):
        tmp[pl.ds(0, L)] = jnp.full((L,), -1, jnp.int32)
        tmp[pl.ds(L, L)] = jnp.full((L,), NB + 2, jnp.int32)
        tmp[pl.ds(1, L)] = ks
        prev = tmp[pl.ds(0, L)]
        nxt = tmp[pl.ds(2, L)]
        chg = ks != prev
        endm = ks != nxt
        run_start = plsc.cummax(jnp.where(chg, lanes, 0))
        return run_start, endm

    def h1(j, _):
        pk = pkb[pl.ds(j * L, L)]
        kk = pk >> SHIFT
        ks, pks = plsc.sort_key_val(kk, pk)
        pkb[pl.ds(j * L, L)] = pks
        run_start, endm = _runs(ks)
        rlen = lanes - run_start + 1
        plsc.addupdate_scatter(hist, [ks], rlen, mask=endm)
        return 0

    lax.fori_loop(0, nv, h1, 0)

    def pfx(i, carry):
        h = hist[pl.ds(i * L, L)]
        c = plsc.cumsum(h)
        ex = carry + c - h
        offs[pl.ds(i * L, L)] = ex
        offs_w[pl.ds(i * L, L)] = ex
        return carry + jnp.max(c)

    lax.fori_loop(0, OFFP // L, pfx, jnp.int32(0))

    def p2(j, _):
        pks = pkb[pl.ds(j * L, L)]
        ks = pks >> SHIFT
        run_start, endm = _runs(ks)
        basev = plsc.load_gather(offs, [ks])
        plsc.store_scatter(pkb2, [basev + lanes - run_start], pks)
        plsc.addupdate_scatter(offs, [ks], lanes - run_start + 1, mask=endm)
        return 0

    lax.fori_loop(0, nv, p2, 0)

    npad = ((rp + K - 1) // K) * K
    nsb = npad // K

    def kp(j, _):
        plsc.store_scatter(pkb2, [rp + j * L + lanes], dummy)
        return 0

    lax.fori_loop(0, (npad - rp) // L, kp, 0)
    plsc.store_scatter(offs_w, [jnp.full((L,), NB + 1, jnp.int32)],
                       jnp.zeros((L,), jnp.int32) + npad, mask=lanes == 0)

    def flushf(b, _):
        pltpu.async_copy(pkb2.at[pl.ds(b * K, K)],
                         pkl.at[pl.ds(pl.multiple_of(w * CAPR + b * K, 8), K)],
                         sem0)
        return 0

    lax.fori_loop(0, nsb, flushf, 0)

    def flushw(b, _):
        pltpu.make_async_copy(
            pkb2.at[pl.ds(b * K, K)],
            pkl.at[pl.ds(pl.multiple_of(w * CAPR + b * K, 8), K)], sem0).wait()
        return 0

    lax.fori_loop(0, nsb, flushw, 0)
    pltpu.sync_copy(offs_w, offsl.at[pl.ds(pl.multiple_of(w * OFFP, 8), OFFP)])
    cnt_v[...] = jnp.zeros((L,), jnp.int32) + (nsb | (ov << 8))
    pltpu.sync_copy(cnt_v, nblk.at[pl.ds(pl.multiple_of(w * L, 8), L)])


def _partition(src, dst):
    return pl.kernel(
        _part_body,
        out_type=[
            jax.ShapeDtypeStruct((NW * CAPR,), jnp.int32),
            jax.ShapeDtypeStruct((NW * L,), jnp.int32),
            jax.ShapeDtypeStruct((NW * OFFP,), jnp.int32),
        ],
        mesh=_mesh(),
        compiler_params=pltpu.CompilerParams(needs_layout_passes=False),
        scratch_types=[
            pltpu.VMEM((CHUNK,), jnp.int32),
            pltpu.VMEM((CHUNK,), jnp.int32),
            pltpu.VMEM((CHUNK,), jnp.int32),
            pltpu.VMEM((CHUNK,), jnp.int32),
            pltpu.VMEM((SORT_CAP + K + CHUNK + L,), jnp.int32),
            pltpu.VMEM((SORT_CAP + K + L,), jnp.int32),
            pltpu.VMEM((OFFP,), jnp.int32),
            pltpu.VMEM((OFFP,), jnp.int32),
            pltpu.VMEM((OFFP,), jnp.int32),
            pltpu.VMEM((3 * L,), jnp.int32),
            pltpu.VMEM((L,), jnp.int32),
            pltpu.SemaphoreType.DMA,
            pltpu.SemaphoreType.DMA,
        ],
    )(src, dst)


def _seg_body(x_hbm, pkl, nblk, offsl, outf, acc1, pk_stage, pk_v, idx_a,
              idx_b, rows_a, rows_b, offs_v, cnt_v, sem_a, sem_b, *, fixup):
    w = _wid()
    pltpu.sync_copy(nblk.at[pl.ds(pl.multiple_of(w * L, 8), L)], cnt_v)
    both = jnp.max(cnt_v[...])
    nsb = both & 0xFF
    nov = both >> 8
    pltpu.sync_copy(offsl.at[pl.ds(pl.multiple_of(w * OFFP, 8), OFFP)], offs_v)

    init = NEG if fixup else 0.0

    @plsc.parallel_loop(0, (NB + 1) * D // L, unroll=4)
    def ini(i):
        acc1[pl.ds(i * L, L)] = jnp.full((L,), init, jnp.float32)

    pltpu.sync_copy(pkl.at[pl.ds(pl.multiple_of(w * CAPR, 8), SB * K)], pk_stage)

    lanes = lax.iota(jnp.int32, L)
    zero_v = jnp.zeros((L,), jnp.int32)

    def _fire_rows(pkref, boff, idx_v, rows_v, sem):
        @plsc.parallel_loop(0, K // L, unroll=4)
        def unp(j):
            v = pkref[pl.ds(boff + j * L, L)]
            idx_v[pl.ds(j * L, L)] = v & ((1 << SHIFT) - 1)

        for j in range(K // KSUB):
            pltpu.async_copy(x_hbm.at[idx_v.at[pl.ds(j * KSUB, KSUB)]],
                             rows_v.at[pl.ds(j * KSUB, KSUB)], sem)

    def _wait_rows(idx_v, rows_v, sem):
        for j in range(K // KSUB):
            pltpu.make_async_copy(x_hbm.at[idx_v.at[pl.ds(j * KSUB, KSUB)]],
                                  rows_v.at[pl.ds(j * KSUB, KSUB)], sem).wait()

    def _drain_sorted(pkref, boff, rows_v, b):
        l_first = jnp.max(plsc.load_gather(pkref, [boff + zero_v])) >> SHIFT
        l_last = jnp.max(plsc.load_gather(pkref, [boff + K - 1 + zero_v])) >> SHIFT
        blo = b * K
        bhi = blo + K

        def node(l, _):
            s = jnp.max(plsc.load_gather(offs_v, [l + zero_v]))
            e = jnp.max(plsc.load_gather(offs_v, [l + 1 + zero_v]))
            s2 = jnp.maximum(s, blo) - blo
            e2 = jnp.minimum(e, bhi) - blo
            iis = [l * D + f * L + lanes for f in range(D // L)]
            accs = [plsc.load_gather(acc1, [ii]) for ii in iis]

            def ee(k, accs_c):
                rrs = [rows_v[k, pl.ds(f * L, L)] for f in range(D // L)]
                return tuple(jnp.maximum(a, rr) for a, rr in zip(accs_c, rrs))

            accs = plsc.parallel_loop(s2, e2, unroll=2, carry=tuple(accs))(ee)
            for ii, a in zip(iis, accs):
                plsc.store_scatter(acc1, [ii], a)
            return 0

        lax.fori_loop(l_first, l_last + 1, node, 0)

    nsb2 = jnp.minimum(nsb, SB)

    @pl.when(nsb2 > 0)
    def _():
        _fire_rows(pk_stage, 0, idx_a, rows_a, sem_a)

    def pair(p, _):
        ba = 2 * p
        bb = 2 * p + 1
        _wait_rows(idx_a, rows_a, sem_a)

        @pl.when(bb < nsb2)
        def _():
            _fire_rows(pk_stage, bb * K, idx_b, rows_b, sem_b)

        _drain_sorted(pk_stage, ba * K, rows_a, ba)

        @pl.when(bb < nsb2)
        def _():
            _wait_rows(idx_b, rows_b, sem_b)

            @pl.when(bb + 1 < nsb2)
            def _():
                _fire_rows(pk_stage, (bb + 1) * K, idx_a, rows_a, sem_a)

            _drain_sorted(pk_stage, bb * K, rows_b, bb)

        return 0

    lax.fori_loop(0, (nsb2 + 1) // 2, pair, 0)

    def batch_hbm(b, _):
        pltpu.sync_copy(pkl.at[pl.ds(pl.multiple_of(w * CAPR + b * K, 8), K)],
                        pk_v)
        _fire_rows(pk_v, 0, idx_a, rows_a, sem_a)
        _wait_rows(idx_a, rows_a, sem_a)
        _drain_sorted(pk_v, 0, rows_a, b)
        return 0

    lax.fori_loop(SB, nsb, batch_hbm, 0)

    def batch_ov(b, _):
        pltpu.sync_copy(
            pkl.at[pl.ds(pl.multiple_of(w * CAPR + OVBASE + b * K, 8), K)],
            pk_v)
        _fire_rows(pk_v, 0, idx_a, rows_a, sem_a)
        _wait_rows(idx_a, rows_a, sem_a)

        def edge(k, _):
            lsp = plsc.load_gather(pk_v, [k + zero_v]) >> SHIFT
            base = lsp * D
            iis = [base + f * L + lanes for f in range(D // L)]
            accs = [plsc.load_gather(acc1, [ii]) for ii in iis]
            rrs = [rows_a[k, pl.ds(f * L, L)] for f in range(D // L)]
            for ii, a, rr in zip(iis, accs, rrs):
                plsc.store_scatter(acc1, [ii], jnp.maximum(a, rr))
            return 0

        lax.fori_loop(0, K, edge, 0)
        return 0

    lax.fori_loop(0, nov, batch_ov, 0)

    if fixup:
        @plsc.parallel_loop(0, NB * D // L, unroll=4)
        def fix(i):
            v = acc1[pl.ds(i * L, L)]
            acc1[pl.ds(i * L, L)] = jnp.where(v == NEG, 0.0, v)

    pltpu.sync_copy(acc1.at[pl.ds(0, NB * D)],
                    outf.at[pl.ds(pl.multiple_of(w * NB * D, 8), NB * D)])


def _segmax(x2d, pkl, nblk, offsl, fixup):
    out = pl.kernel(
        functools.partial(_seg_body, fixup=fixup),
        out_type=jax.ShapeDtypeStruct((NPAD * D,), jnp.float32),
        mesh=_mesh(),
        compiler_params=pltpu.CompilerParams(needs_layout_passes=False),
        scratch_types=[
            pltpu.VMEM(((NB + 1) * D,), jnp.float32),
            pltpu.VMEM((SB * K,), jnp.int32),
            pltpu.VMEM((K,), jnp.int32),
            pltpu.VMEM((K,), jnp.int32),
            pltpu.VMEM((K,), jnp.int32),
            pltpu.VMEM((K, D), jnp.float32),
            pltpu.VMEM((K, D), jnp.float32),
            pltpu.VMEM((OFFP,), jnp.int32),
            pltpu.VMEM((L,), jnp.int32),
            pltpu.SemaphoreType.DMA,
            pltpu.SemaphoreType.DMA,
        ],
    )(x2d, pkl, nblk, offsl)
    return out.reshape(NPAD, D)


_ROWS = 1000


def _lin_body(a_ref, x_ref, wl_ref, bl_ref, wr_ref, o_ref):
    acc = lax.dot_general(a_ref[...], wl_ref[...], (((1,), (1,)), ((), ())),
                          preferred_element_type=jnp.float32)
    acc += lax.dot_general(x_ref[...], wr_ref[...], (((1,), (1,)), ((), ())),
                           preferred_element_type=jnp.float32)
    acc += bl_ref[...]
    o_ref[...] = jnp.maximum(acc, 0.0)


def _fused_lin(a, x, wl, bl, wr):
    return pl.pallas_call(
        _lin_body,
        grid=(N // _ROWS,),
        in_specs=[
            pl.BlockSpec((_ROWS, D), lambda i: (i, 0)),
            pl.BlockSpec((_ROWS, D), lambda i: (i, 0)),
            pl.BlockSpec((D, D), lambda i: (0, 0)),
            pl.BlockSpec((1, D), lambda i: (0, 0)),
            pl.BlockSpec((D, D), lambda i: (0, 0)),
        ],
        out_specs=pl.BlockSpec((_ROWS, D), lambda i: (i, 0)),
        out_shape=jax.ShapeDtypeStruct((N, D), jnp.float32),
    )(a, x, wl, bl.reshape(1, D), wr)


def _final_body(a_ref, x_ref, wl_ref, bl_ref, wr_ref, wf_ref, bf_ref, o_ref):
    acc = lax.dot_general(a_ref[...], wl_ref[...], (((1,), (1,)), ((), ())),
                          preferred_element_type=jnp.float32)
    acc += lax.dot_general(x_ref[...], wr_ref[...], (((1,), (1,)), ((), ())),
                           preferred_element_type=jnp.float32)
    h = jnp.maximum(acc + bl_ref[...], 0.0)
    o_ref[...] = lax.dot_general(h, wf_ref[...], (((1,), (1,)), ((), ())),
                                 preferred_element_type=jnp.float32) + bf_ref[...]


def _final_lin(a, x, wl, bl, wr, wf, bf):
    return pl.pallas_call(
        _final_body,
        grid=(N // _ROWS,),
        in_specs=[
            pl.BlockSpec((_ROWS, D), lambda i: (i, 0)),
            pl.BlockSpec((_ROWS, D), lambda i: (i, 0)),
            pl.BlockSpec((D, D), lambda i: (0, 0)),
            pl.BlockSpec((1, D), lambda i: (0, 0)),
            pl.BlockSpec((D, D), lambda i: (0, 0)),
            pl.BlockSpec((D, D), lambda i: (0, 0)),
            pl.BlockSpec((1, D), lambda i: (0, 0)),
        ],
        out_specs=pl.BlockSpec((_ROWS, D), lambda i: (i, 0)),
        out_shape=jax.ShapeDtypeStruct((N, D), jnp.float32),
    )(a, x, wl, bl.reshape(1, D), wr, wf, bf.reshape(1, D))


def kernel(x, edge_index, Wl0, bl0, Wr0, Wl1, bl1, Wr1, W, b):
    src = edge_index[0].astype(jnp.int32)
    dst = edge_index[1].astype(jnp.int32)
    pkl, nblk, offsl = _partition(src, dst)
    a0 = _segmax(x, pkl, nblk, offsl, fixup=True)
    h1 = _fused_lin(a0, x, Wl0, bl0, Wr0)
    a1 = _segmax(h1, pkl, nblk, offsl, fixup=False)
    out = _final_lin(a1, h1, Wl1, bl1, Wr1, W, b)
    return out
